# baseline (speedup 1.0000x reference)
import functools
import math

import jax
import jax.numpy as jnp
from jax import lax
from jax.experimental import pallas as pl
from jax.experimental.pallas import tpu as pltpu
from jax.experimental.pallas import tpu_sc as plsc

_SCALE = 10.0
_MARGIN = 0.35
_SM = _SCALE * _MARGIN
_EM1 = math.expm1(-_SM)


def _make_sc_gather(n_rows, n_cls):
    info = plsc.get_sparse_core_info()
    nw = info.num_cores * info.num_subcores
    rows_per_w = n_rows // nw
    n_batches = rows_per_w // 16
    mesh = plsc.VectorSubcoreMesh(core_axis_name="c", subcore_axis_name="s")

    @functools.partial(
        pl.kernel, mesh=mesh,
        out_type=jax.ShapeDtypeStruct((n_rows, 128), jnp.float32),
        scratch_types=[
            pltpu.VMEM((rows_per_w,), jnp.int32),
            pltpu.VMEM((rows_per_w, 128), jnp.float32),
            pltpu.SemaphoreType.DMA,
        ],
    )
    def sc_gather(table_hbm, widx_hbm, out_hbm, idx_v, win_v, sem):
        wid = lax.axis_index("s") * info.num_cores + lax.axis_index("c")
        base = wid * rows_per_w
        pltpu.sync_copy(widx_hbm.at[pl.ds(base, rows_per_w)], idx_v)
        pltpu.async_copy(table_hbm.at[idx_v], win_v, sem).wait()
        pltpu.sync_copy(win_v, out_hbm.at[pl.ds(base, rows_per_w)])

    return sc_gather


def _lse_kernel(x_ref, win_ref, off_ref, out_ref, m_ref, s_ref, *,
                num_rows, num_classes):
    r = pl.program_id(0)
    c = pl.program_id(1)
    nc = pl.num_programs(1)
    w = x_ref.shape[1]

    @pl.when(c == 0)
    def _init():
        m_ref[...] = jnp.full_like(m_ref, -jnp.inf)
        s_ref[...] = jnp.zeros_like(s_ref)

    def _update(masked):
        x = x_ref[...]
        if masked:
            lcol = jax.lax.broadcasted_iota(jnp.int32, x.shape, 1)
            x = jnp.where(lcol < num_classes - c * w, x, -jnp.inf)
        bmax = jnp.max(x, axis=1, keepdims=True) * _SCALE
        m_old = m_ref[...]
        m_new = jnp.maximum(m_old, bmax)
        bsum = jnp.sum(jnp.exp(x * _SCALE - m_new), axis=1, keepdims=True)
        s_ref[...] = s_ref[...] * jnp.exp(m_old - m_new) + bsum
        m_ref[...] = m_new

    ragged = num_classes % w != 0

    @pl.when(c < nc - 1)
    def _full():
        _update(masked=False)

    @pl.when(c == nc - 1)
    def _last():
        _update(masked=ragged)

    @pl.when(c == nc - 1)
    def _finish():
        lse = m_ref[...] + jnp.log(s_ref[...])
        win = win_ref[...]
        lane = jax.lax.broadcasted_iota(jnp.int32, win.shape, 1)
        tval = jnp.sum(jnp.where(lane == off_ref[...], win, 0.0),
                       axis=1, keepdims=True)
        a_t = tval * _SCALE
        row_loss = lse + jnp.log(1.0 + _EM1 * jnp.exp(a_t - lse)) - a_t + _SM
        part = jnp.sum(row_loss, keepdims=True).reshape(1, 1) * (1.0 / num_rows)

        @pl.when(r == 0)
        def _():
            out_ref[...] = part

        @pl.when(r != 0)
        def _():
            out_ref[...] = out_ref[...] + part


def kernel(input, target):
    n_rows, n_cls = input.shape
    tgt = target.astype(jnp.int32)

    table = input.reshape(n_rows * n_cls // 128, 128)
    rows = jnp.arange(n_rows, dtype=jnp.int32)
    flat = rows * n_cls + tgt
    widx = flat >> 7
    off2d = (flat & 127).reshape(n_rows, 1)
    wins = _make_sc_gather(n_rows, n_cls)(table, widx)

    block_r = min(256, n_rows)
    block_w = min(4096, n_cls)
    grid = (pl.cdiv(n_rows, block_r), pl.cdiv(n_cls, block_w))

    out = pl.pallas_call(
        functools.partial(_lse_kernel, num_rows=n_rows, num_classes=n_cls),
        grid=grid,
        in_specs=[
            pl.BlockSpec((block_r, block_w), lambda r, c: (r, c)),
            pl.BlockSpec((block_r, 128), lambda r, c: (r, 0)),
            pl.BlockSpec((block_r, 1), lambda r, c: (r, 0)),
        ],
        out_specs=pl.BlockSpec((1, 1), lambda r, c: (0, 0)),
        out_shape=jax.ShapeDtypeStruct((1, 1), jnp.float32),
        scratch_shapes=[
            pltpu.VMEM((block_r, 1), jnp.float32),
            pltpu.VMEM((block_r, 1), jnp.float32),
        ],
    )(input, wins, off2d)
    return out[0, 0]


# speedup vs baseline: 2.0260x; 2.0260x over previous
import functools
import math

import jax
import jax.numpy as jnp
from jax import lax
from jax.experimental import pallas as pl
from jax.experimental.pallas import tpu as pltpu

_SCALE = 10.0
_MARGIN = 0.35
_SM = _SCALE * _MARGIN
_EM1 = math.expm1(-_SM)


def _lse_kernel(col0_ref, x_ref, xany_ref, off_ref, out_ref,
                m_ref, s_ref, win_ref, sem, *, num_rows, num_classes):
    r = pl.program_id(0)
    c = pl.program_id(1)
    nc = pl.num_programs(1)
    w = x_ref.shape[1]
    br = x_ref.shape[0]

    @pl.when(c == 0)
    def _init():
        m_ref[...] = jnp.full_like(m_ref, -jnp.inf)
        s_ref[...] = jnp.zeros_like(s_ref)

        def _fire(i, _):
            row = r * br + i
            row8 = pl.multiple_of((row // 8) * 8, 8)
            pltpu.make_async_copy(
                xany_ref.at[pl.ds(row8, 8),
                            pl.ds(pl.multiple_of(col0_ref[row], 128), 128)],
                win_ref.at[i], sem).start()
            return _

        lax.fori_loop(0, br, _fire, None)

    def _update(masked):
        x = x_ref[...]
        if masked:
            lcol = jax.lax.broadcasted_iota(jnp.int32, x.shape, 1)
            x = jnp.where(lcol < num_classes - c * w, x, -jnp.inf)
        bmax = jnp.max(x, axis=1, keepdims=True) * _SCALE
        m_old = m_ref[...]
        m_new = jnp.maximum(m_old, bmax)
        bsum = jnp.sum(jnp.exp(x * _SCALE - m_new), axis=1, keepdims=True)
        s_ref[...] = s_ref[...] * jnp.exp(m_old - m_new) + bsum
        m_ref[...] = m_new

    ragged = num_classes % w != 0

    @pl.when(c < nc - 1)
    def _full():
        _update(masked=False)

    @pl.when(c == nc - 1)
    def _last():
        _update(masked=ragged)

    @pl.when(c == nc - 1)
    def _finish():
        def _drain(i, _):
            row = r * br + i
            row8 = pl.multiple_of((row // 8) * 8, 8)
            pltpu.make_async_copy(
                xany_ref.at[pl.ds(row8, 8),
                            pl.ds(pl.multiple_of(col0_ref[row], 128), 128)],
                win_ref.at[i], sem).wait()
            return _

        lax.fori_loop(0, br, _drain, None)

        lse = m_ref[...] + jnp.log(s_ref[...])
        win = win_ref[...]
        sub = jax.lax.broadcasted_iota(jnp.int32, win.shape, 1)
        myrow = jax.lax.broadcasted_iota(jnp.int32, win.shape, 0) & 7
        lane = jax.lax.broadcasted_iota(jnp.int32, win.shape, 2)
        sel = (sub == myrow) & (lane == off_ref[...].reshape(-1, 1, 1))
        tval = jnp.sum(jnp.where(sel, win, 0.0), axis=(1, 2), keepdims=False)
        tval = tval.reshape(-1, 1)
        a_t = tval * _SCALE
        row_loss = lse + jnp.log(1.0 + _EM1 * jnp.exp(a_t - lse)) - a_t + _SM
        part = jnp.sum(row_loss, keepdims=True).reshape(1, 1) * (1.0 / num_rows)

        @pl.when(r == 0)
        def _():
            out_ref[...] = part

        @pl.when(r != 0)
        def _():
            out_ref[...] = out_ref[...] + part


def kernel(input, target):
    n_rows, n_cls = input.shape
    tgt = target.astype(jnp.int32)

    col0 = tgt & ~jnp.int32(127)
    off2d = (tgt - col0).reshape(n_rows, 1)

    block_r = min(256, n_rows)
    block_w = min(4096, n_cls)
    grid = (pl.cdiv(n_rows, block_r), pl.cdiv(n_cls, block_w))

    out = pl.pallas_call(
        functools.partial(_lse_kernel, num_rows=n_rows, num_classes=n_cls),
        grid_spec=pltpu.PrefetchScalarGridSpec(
            num_scalar_prefetch=1,
            grid=grid,
            in_specs=[
                pl.BlockSpec((block_r, block_w), lambda r, c, s: (r, c)),
                pl.BlockSpec(memory_space=pl.ANY),
                pl.BlockSpec((block_r, 1), lambda r, c, s: (r, 0)),
            ],
            out_specs=pl.BlockSpec((1, 1), lambda r, c, s: (0, 0)),
            scratch_shapes=[
                pltpu.VMEM((block_r, 1), jnp.float32),
                pltpu.VMEM((block_r, 1), jnp.float32),
                pltpu.VMEM((block_r, 8, 128), jnp.float32),
                pltpu.SemaphoreType.DMA,
            ],
        ),
        out_shape=jax.ShapeDtypeStruct((1, 1), jnp.float32),
    )(col0, input, input, off2d)
    return out[0, 0]


# speedup vs baseline: 6.5371x; 3.2266x over previous
import functools
import math

import jax
import jax.numpy as jnp
from jax.experimental import pallas as pl
from jax.experimental.pallas import tpu as pltpu

_SCALE = 10.0
_MARGIN = 0.35
_SM = _SCALE * _MARGIN
_EM1 = math.expm1(-_SM)


def _lse_kernel(x_ref, tgt_ref, out_ref, m_ref, s_ref, t_ref, *,
                num_rows, num_classes):
    c = pl.program_id(0)
    nc = pl.num_programs(0)
    w = x_ref.shape[0]

    @pl.when(c == 0)
    def _init():
        m_ref[...] = jnp.full_like(m_ref, -jnp.inf)
        s_ref[...] = jnp.zeros_like(s_ref)
        t_ref[...] = jnp.zeros_like(t_ref)

    def _update(masked):
        x = x_ref[...]
        cls = jax.lax.broadcasted_iota(jnp.int32, x.shape, 0) + c * w
        is_t = cls == tgt_ref[...]
        t_ref[...] = t_ref[...] + jnp.sum(
            jnp.where(is_t, x, 0.0), axis=0, keepdims=True)
        if masked:
            x = jnp.where(cls < num_classes, x, -jnp.inf)
        bmax = jnp.max(x, axis=0, keepdims=True) * _SCALE
        m_old = m_ref[...]
        m_new = jnp.maximum(m_old, bmax)
        bsum = jnp.sum(jnp.exp(x * _SCALE - m_new), axis=0, keepdims=True)
        s_ref[...] = s_ref[...] * jnp.exp(m_old - m_new) + bsum
        m_ref[...] = m_new

    ragged = num_classes % w != 0

    @pl.when(c < nc - 1)
    def _full():
        _update(masked=False)

    @pl.when(c == nc - 1)
    def _last():
        _update(masked=ragged)

    @pl.when(c == nc - 1)
    def _finish():
        lse = m_ref[...] + jnp.log(s_ref[...])
        a_t = t_ref[...] * _SCALE
        row_loss = lse + jnp.log(1.0 + _EM1 * jnp.exp(a_t - lse)) - a_t + _SM
        out_ref[...] = jnp.sum(row_loss, keepdims=True).reshape(1, 1) * (
            1.0 / num_rows)


def kernel(input, target):
    n_rows, n_cls = input.shape
    xt = input.T
    tgt2d = target.astype(jnp.int32).reshape(1, n_rows)

    block_c = min(2048, n_cls)
    grid = (pl.cdiv(n_cls, block_c),)

    out = pl.pallas_call(
        functools.partial(_lse_kernel, num_rows=n_rows, num_classes=n_cls),
        grid=grid,
        in_specs=[
            pl.BlockSpec((block_c, n_rows), lambda c: (c, 0)),
            pl.BlockSpec((1, n_rows), lambda c: (0, 0)),
        ],
        out_specs=pl.BlockSpec((1, 1), lambda c: (0, 0)),
        out_shape=jax.ShapeDtypeStruct((1, 1), jnp.float32),
        scratch_shapes=[
            pltpu.VMEM((1, n_rows), jnp.float32),
            pltpu.VMEM((1, n_rows), jnp.float32),
            pltpu.VMEM((1, n_rows), jnp.float32),
        ],
    )(xt, tgt2d)
    return out[0, 0]


# speedup vs baseline: 6.6227x; 1.0131x over previous
import functools
import math

import jax
import jax.numpy as jnp
from jax.experimental import pallas as pl
from jax.experimental.pallas import tpu as pltpu

_SCALE = 10.0
_MARGIN = 0.35
_SM = _SCALE * _MARGIN
_EM1 = math.expm1(-_SM)


def _lse_kernel(x_ref, tgt_ref, out_ref, m_ref, s_ref, t_ref, *,
                num_rows, num_classes):
    c = pl.program_id(0)
    nc = pl.num_programs(0)
    w = x_ref.shape[0]

    @pl.when(c == 0)
    def _init():
        m_ref[...] = jnp.full_like(m_ref, -jnp.inf)
        s_ref[...] = jnp.zeros_like(s_ref)
        t_ref[...] = jnp.zeros_like(t_ref)

    def _update(masked):
        x = x_ref[...]
        cls = jax.lax.broadcasted_iota(jnp.int32, x.shape, 0) + c * w
        is_t = cls == tgt_ref[...]
        t_ref[...] = t_ref[...] + jnp.sum(
            jnp.where(is_t, x, 0.0), axis=0, keepdims=True)
        if masked:
            x = jnp.where(cls < num_classes, x, -jnp.inf)
        bmax = jnp.max(x, axis=0, keepdims=True) * _SCALE
        m_old = m_ref[...]
        m_new = jnp.maximum(m_old, bmax)
        bsum = jnp.sum(jnp.exp(x * _SCALE - m_new), axis=0, keepdims=True)
        s_ref[...] = s_ref[...] * jnp.exp(m_old - m_new) + bsum
        m_ref[...] = m_new

    ragged = num_classes % w != 0

    @pl.when(c < nc - 1)
    def _full():
        _update(masked=False)

    @pl.when(c == nc - 1)
    def _last():
        _update(masked=ragged)

    @pl.when(c == nc - 1)
    def _finish():
        lse = m_ref[...] + jnp.log(s_ref[...])
        a_t = t_ref[...] * _SCALE
        row_loss = lse + jnp.log(1.0 + _EM1 * jnp.exp(a_t - lse)) - a_t + _SM
        out_ref[...] = jnp.sum(row_loss, keepdims=True).reshape(1, 1) * (
            1.0 / num_rows)


def kernel(input, target):
    n_rows, n_cls = input.shape
    xt = input.T
    tgt2d = target.astype(jnp.int32).reshape(1, n_rows)

    block_c = min(4096, n_cls)
    grid = (pl.cdiv(n_cls, block_c),)

    out = pl.pallas_call(
        functools.partial(_lse_kernel, num_rows=n_rows, num_classes=n_cls),
        grid=grid,
        in_specs=[
            pl.BlockSpec((block_c, n_rows), lambda c: (c, 0)),
            pl.BlockSpec((1, n_rows), lambda c: (0, 0)),
        ],
        out_specs=pl.BlockSpec((1, 1), lambda c: (0, 0)),
        out_shape=jax.ShapeDtypeStruct((1, 1), jnp.float32),
        scratch_shapes=[
            pltpu.VMEM((1, n_rows), jnp.float32),
            pltpu.VMEM((1, n_rows), jnp.float32),
            pltpu.VMEM((1, n_rows), jnp.float32),
        ],
    )(xt, tgt2d)
    return out[0, 0]


# speedup vs baseline: 6.8792x; 1.0387x over previous
import functools
import math

import jax
import jax.numpy as jnp
from jax.experimental import pallas as pl
from jax.experimental.pallas import tpu as pltpu

_SCALE = 10.0
_MARGIN = 0.35
_SM = _SCALE * _MARGIN
_EM1 = math.expm1(-_SM)
_LOG2E = math.log2(math.e)
_K2 = _SCALE * _LOG2E
_LN2 = math.log(2.0)


def _lse_kernel(x_ref, tgt_ref, out_ref, m_ref, s_ref, t_ref, *,
                num_rows, num_classes):
    c = pl.program_id(0)
    nc = pl.num_programs(0)
    w = x_ref.shape[0]

    @pl.when(c == 0)
    def _init():
        m_ref[...] = jnp.full_like(m_ref, -jnp.inf)
        s_ref[...] = jnp.zeros_like(s_ref)
        t_ref[...] = jnp.zeros_like(t_ref)

    def _update(masked):
        x = x_ref[...]
        cls = jax.lax.broadcasted_iota(jnp.int32, x.shape, 0) + c * w
        is_t = cls == tgt_ref[...]
        t_ref[...] = t_ref[...] + jnp.sum(
            jnp.where(is_t, x, 0.0), axis=0, keepdims=True)
        if masked:
            x = jnp.where(cls < num_classes, x, -jnp.inf)
        bmax = jnp.max(x, axis=0, keepdims=True) * _K2
        m_old = m_ref[...]
        m_new = jnp.maximum(m_old, bmax)
        bsum = jnp.sum(jnp.exp2(x * _K2 - m_new), axis=0, keepdims=True)
        s_ref[...] = s_ref[...] * jnp.exp2(m_old - m_new) + bsum
        m_ref[...] = m_new

    ragged = num_classes % w != 0

    @pl.when(c < nc - 1)
    def _full():
        _update(masked=False)

    @pl.when(c == nc - 1)
    def _last():
        _update(masked=ragged)

    @pl.when(c == nc - 1)
    def _finish():
        lse = m_ref[...] * _LN2 + jnp.log(s_ref[...])
        a_t = t_ref[...] * _SCALE
        row_loss = lse + jnp.log(1.0 + _EM1 * jnp.exp(a_t - lse)) - a_t + _SM
        out_ref[...] = jnp.sum(row_loss, keepdims=True).reshape(1, 1) * (
            1.0 / num_rows)


def kernel(input, target):
    n_rows, n_cls = input.shape
    xt = input.T
    tgt2d = target.astype(jnp.int32).reshape(1, n_rows)

    block_c = min(4096, n_cls)
    grid = (pl.cdiv(n_cls, block_c),)

    out = pl.pallas_call(
        functools.partial(_lse_kernel, num_rows=n_rows, num_classes=n_cls),
        grid=grid,
        in_specs=[
            pl.BlockSpec((block_c, n_rows), lambda c: (c, 0)),
            pl.BlockSpec((1, n_rows), lambda c: (0, 0)),
        ],
        out_specs=pl.BlockSpec((1, 1), lambda c: (0, 0)),
        out_shape=jax.ShapeDtypeStruct((1, 1), jnp.float32),
        scratch_shapes=[
            pltpu.VMEM((1, n_rows), jnp.float32),
            pltpu.VMEM((1, n_rows), jnp.float32),
            pltpu.VMEM((1, n_rows), jnp.float32),
        ],
    )(xt, tgt2d)
    return out[0, 0]


# speedup vs baseline: 7.2634x; 1.0558x over previous
import functools
import math

import jax
import jax.numpy as jnp
from jax import lax
from jax.experimental import pallas as pl
from jax.experimental.pallas import tpu as pltpu

_SCALE = 10.0
_MARGIN = 0.35
_SM = _SCALE * _MARGIN
_EM1 = math.expm1(-_SM)
_LOG2E = math.log2(math.e)
_K2 = _SCALE * _LOG2E
_LN2 = math.log(2.0)


def _lse_kernel(tgt8_ref, x_ref, xany_ref, sub_ref, out_ref,
                m_ref, s_ref, win_ref, sem, *, num_rows, num_classes):
    c = pl.program_id(0)
    nc = pl.num_programs(0)
    w = x_ref.shape[0]

    @pl.when(c == 0)
    def _init():
        m_ref[...] = jnp.full_like(m_ref, -jnp.inf)
        s_ref[...] = jnp.zeros_like(s_ref)

        def _fire(b, _):
            b0 = pl.multiple_of((b // 128) * 128, 128)
            pltpu.make_async_copy(
                xany_ref.at[pl.ds(pl.multiple_of(tgt8_ref[b], 8), 8),
                            pl.ds(b0, 128)],
                win_ref.at[b], sem).start()
            return _

        lax.fori_loop(0, num_rows, _fire, None)

    def _update(masked):
        x = x_ref[...]
        if masked:
            cls = jax.lax.broadcasted_iota(jnp.int32, x.shape, 0) + c * w
            x = jnp.where(cls < num_classes, x, -jnp.inf)
        bmax = jnp.max(x, axis=0, keepdims=True) * _K2
        m_old = m_ref[...]
        m_new = jnp.maximum(m_old, bmax)
        bsum = jnp.sum(jnp.exp2(x * _K2 - m_new), axis=0, keepdims=True)
        s_ref[...] = s_ref[...] * jnp.exp2(m_old - m_new) + bsum
        m_ref[...] = m_new

    ragged = num_classes % w != 0

    @pl.when(c < nc - 1)
    def _full():
        _update(masked=False)

    @pl.when(c == nc - 1)
    def _last():
        _update(masked=ragged)

    @pl.when(c == nc - 1)
    def _finish():
        def _drain(b, _):
            b0 = pl.multiple_of((b // 128) * 128, 128)
            pltpu.make_async_copy(
                xany_ref.at[pl.ds(pl.multiple_of(tgt8_ref[b], 8), 8),
                            pl.ds(b0, 128)],
                win_ref.at[b], sem).wait()
            return _

        lax.fori_loop(0, num_rows, _drain, None)

        win = win_ref[...]
        bi = jax.lax.broadcasted_iota(jnp.int32, win.shape, 0)
        sub = jax.lax.broadcasted_iota(jnp.int32, win.shape, 1)
        lane = jax.lax.broadcasted_iota(jnp.int32, win.shape, 2)
        sel = (sub == sub_ref[...].reshape(-1, 1, 1)) & (lane == (bi & 127))
        tval = jnp.sum(jnp.where(sel, win, 0.0), axis=(1, 2))
        a_t = tval.reshape(1, -1) * _SCALE

        lse = m_ref[...] * _LN2 + jnp.log(s_ref[...])
        row_loss = lse + jnp.log(1.0 + _EM1 * jnp.exp(a_t - lse)) - a_t + _SM
        out_ref[...] = jnp.sum(row_loss, keepdims=True).reshape(1, 1) * (
            1.0 / num_rows)


def kernel(input, target):
    n_rows, n_cls = input.shape
    xt = input.T
    tgt = target.astype(jnp.int32)
    tgt8 = tgt & ~jnp.int32(7)
    sub2d = (tgt & 7).reshape(n_rows, 1)

    block_c = min(4096, n_cls)
    grid = (pl.cdiv(n_cls, block_c),)

    out = pl.pallas_call(
        functools.partial(_lse_kernel, num_rows=n_rows, num_classes=n_cls),
        grid_spec=pltpu.PrefetchScalarGridSpec(
            num_scalar_prefetch=1,
            grid=grid,
            in_specs=[
                pl.BlockSpec((block_c, n_rows), lambda c, s: (c, 0)),
                pl.BlockSpec(memory_space=pl.ANY),
                pl.BlockSpec((n_rows, 1), lambda c, s: (0, 0)),
            ],
            out_specs=pl.BlockSpec((1, 1), lambda c, s: (0, 0)),
            scratch_shapes=[
                pltpu.VMEM((1, n_rows), jnp.float32),
                pltpu.VMEM((1, n_rows), jnp.float32),
                pltpu.VMEM((n_rows, 8, 128), jnp.float32),
                pltpu.SemaphoreType.DMA,
            ],
        ),
        out_shape=jax.ShapeDtypeStruct((1, 1), jnp.float32),
    )(tgt8, xt, xt, sub2d)
    return out[0, 0]


# speedup vs baseline: 7.6071x; 1.0473x over previous
import functools
import math

import jax
import jax.numpy as jnp
from jax import lax
from jax.experimental import pallas as pl
from jax.experimental.pallas import tpu as pltpu

_SCALE = 10.0
_MARGIN = 0.35
_SM = _SCALE * _MARGIN
_EM1 = math.expm1(-_SM)
_LOG2E = math.log2(math.e)
_K2 = _SCALE * _LOG2E
_LN2 = math.log(2.0)


def _lse_kernel(tgt8_ref, x_ref, xany_ref, sub_ref, out_ref,
                m_ref, s_ref, win_ref, tval_ref, sem, *,
                num_rows, num_classes, fire_steps, drain_step):
    c = pl.program_id(0)
    nc = pl.num_programs(0)
    w = x_ref.shape[0]
    chunk = num_rows // fire_steps

    @pl.when(c == 0)
    def _init():
        m_ref[...] = jnp.full_like(m_ref, -jnp.inf)
        s_ref[...] = jnp.zeros_like(s_ref)

    @pl.when(c < fire_steps)
    def _fire_chunk():
        def _fire(b, _):
            b0 = pl.multiple_of((b // 128) * 128, 128)
            pltpu.make_async_copy(
                xany_ref.at[pl.ds(pl.multiple_of(tgt8_ref[b], 8), 8),
                            pl.ds(b0, 128)],
                win_ref.at[b], sem).start()
            return _

        lax.fori_loop(c * chunk, (c + 1) * chunk, _fire, None)

    def _update(masked):
        x = x_ref[...]
        if masked:
            cls = jax.lax.broadcasted_iota(jnp.int32, x.shape, 0) + c * w
            x = jnp.where(cls < num_classes, x, -jnp.inf)
        bmax = jnp.max(x, axis=0, keepdims=True) * _K2
        m_old = m_ref[...]
        m_new = jnp.maximum(m_old, bmax)
        bsum = jnp.sum(jnp.exp2(x * _K2 - m_new), axis=0, keepdims=True)
        s_ref[...] = s_ref[...] * jnp.exp2(m_old - m_new) + bsum
        m_ref[...] = m_new

    ragged = num_classes % w != 0

    @pl.when(c < nc - 1)
    def _full():
        _update(masked=False)

    @pl.when(c == nc - 1)
    def _last():
        _update(masked=ragged)

    @pl.when(c == drain_step)
    def _extract():
        def _drain(b, _):
            b0 = pl.multiple_of((b // 128) * 128, 128)
            pltpu.make_async_copy(
                xany_ref.at[pl.ds(pl.multiple_of(tgt8_ref[b], 8), 8),
                            pl.ds(b0, 128)],
                win_ref.at[b], sem).wait()
            return _

        lax.fori_loop(0, num_rows, _drain, None)

        win = win_ref[...]
        bi = jax.lax.broadcasted_iota(jnp.int32, win.shape, 0)
        sub = jax.lax.broadcasted_iota(jnp.int32, win.shape, 1)
        lane = jax.lax.broadcasted_iota(jnp.int32, win.shape, 2)
        sel = (sub == sub_ref[...].reshape(-1, 1, 1)) & (lane == (bi & 127))
        tval = jnp.sum(jnp.where(sel, win, 0.0), axis=(1, 2))
        tval_ref[...] = tval.reshape(1, -1)

    @pl.when(c == nc - 1)
    def _finish():
        a_t = tval_ref[...] * _SCALE
        lse = m_ref[...] * _LN2 + jnp.log(s_ref[...])
        row_loss = lse + jnp.log(1.0 + _EM1 * jnp.exp(a_t - lse)) - a_t + _SM
        out_ref[...] = jnp.sum(row_loss, keepdims=True).reshape(1, 1) * (
            1.0 / num_rows)


def kernel(input, target):
    n_rows, n_cls = input.shape
    xt = input.T
    tgt = target.astype(jnp.int32)
    tgt8 = tgt & ~jnp.int32(7)
    sub2d = (tgt & 7).reshape(n_rows, 1)

    block_c = min(4096, n_cls)
    grid = (pl.cdiv(n_cls, block_c),)
    nc_blocks = grid[0]
    fire_steps = 8 if nc_blocks >= 12 and n_rows % 8 == 0 else 1
    drain_step = nc_blocks - 2 if nc_blocks >= 12 else nc_blocks - 1

    out = pl.pallas_call(
        functools.partial(_lse_kernel, num_rows=n_rows, num_classes=n_cls,
                          fire_steps=fire_steps, drain_step=drain_step),
        grid_spec=pltpu.PrefetchScalarGridSpec(
            num_scalar_prefetch=1,
            grid=grid,
            in_specs=[
                pl.BlockSpec((block_c, n_rows), lambda c, s: (c, 0)),
                pl.BlockSpec(memory_space=pl.ANY),
                pl.BlockSpec((n_rows, 1), lambda c, s: (0, 0)),
            ],
            out_specs=pl.BlockSpec((1, 1), lambda c, s: (0, 0)),
            scratch_shapes=[
                pltpu.VMEM((1, n_rows), jnp.float32),
                pltpu.VMEM((1, n_rows), jnp.float32),
                pltpu.VMEM((n_rows, 8, 128), jnp.float32),
                pltpu.VMEM((1, n_rows), jnp.float32),
                pltpu.SemaphoreType.DMA,
            ],
        ),
        out_shape=jax.ShapeDtypeStruct((1, 1), jnp.float32),
    )(tgt8, xt, xt, sub2d)
    return out[0, 0]
